# Initial kernel scaffold; baseline (speedup 1.0000x reference)
import jax
import jax.numpy as jnp
from jax.experimental import pallas as pl


def kernel(x, edge_index, batch, W_root0, W_nbr0, b0, W_root1, W_nbr1, b1):
    raise NotImplementedError("write your pallas kernel here")


# speedup vs baseline: 4.4039x; 4.4039x over previous
import functools

import jax
import jax.numpy as jnp
from jax import lax
from jax.experimental import pallas as pl
from jax.experimental.pallas import tpu as pltpu
from jax.experimental.pallas import tpu_sc as plsc

NC = 2
NS = 16
NW = NC * NS
CH = 128
G_OUT = 64


def _scatter_sum_sc(h_pad, src_w, dst_w, zeros_blk, n_pad, nch):
    rpt = n_pad // NS
    mesh = plsc.VectorSubcoreMesh(core_axis_name="c", subcore_axis_name="s")

    @functools.partial(
        pl.kernel,
        out_type=jax.ShapeDtypeStruct((NC, n_pad, 128), jnp.float32),
        mesh=mesh,
        scratch_types=[
            pltpu.VMEM((nch, CH), jnp.int32),
            pltpu.VMEM((nch, CH), jnp.int32),
            pltpu.VMEM((CH, 128), jnp.float32),
            pltpu.VMEM_SHARED((n_pad, 128), jnp.float32),
            pltpu.SemaphoreType.DMA,
        ],
    )
    def k(h_hbm, src_hbm, dst_hbm, z_hbm, out_hbm, srcv, dstv, rows, acc, sem):
        c = lax.axis_index("c")
        s = lax.axis_index("s")
        wid = c * NS + s
        pltpu.sync_copy(z_hbm, acc.at[pl.ds(s * rpt, rpt)])
        pltpu.sync_copy(src_hbm.at[wid], srcv)
        pltpu.sync_copy(dst_hbm.at[wid], dstv)
        plsc.subcore_barrier()

        def body(j, carry):
            pltpu.async_copy(h_hbm.at[srcv.at[j]], rows, sem).wait()
            pltpu.sync_copy(rows, acc.at[dstv.at[j]], add=True)
            return carry

        lax.fori_loop(0, nch, body, 0)
        plsc.subcore_barrier()
        pltpu.sync_copy(acc.at[pl.ds(s * rpt, rpt)],
                        out_hbm.at[c, pl.ds(s * rpt, rpt)])

    return k(h_pad, src_w, dst_w, zeros_blk)


def _dense_tc(h_pad, a0, a1, wr, wn, b_row, n_pad, bn):
    def body(h_ref, a0_ref, a1_ref, wr_ref, wn_ref, b_ref, o_ref):
        z = jnp.dot(h_ref[...], wr_ref[...],
                    preferred_element_type=jnp.float32,
                    precision=lax.Precision.HIGHEST)
        z += jnp.dot(a0_ref[...] + a1_ref[...], wn_ref[...],
                     preferred_element_type=jnp.float32,
                     precision=lax.Precision.HIGHEST)
        o_ref[...] = jnp.maximum(z + b_ref[...], 0.0)

    return pl.pallas_call(
        body,
        grid=(n_pad // bn,),
        in_specs=[
            pl.BlockSpec((bn, 128), lambda i: (i, 0)),
            pl.BlockSpec((bn, 128), lambda i: (i, 0)),
            pl.BlockSpec((bn, 128), lambda i: (i, 0)),
            pl.BlockSpec((128, 128), lambda i: (0, 0)),
            pl.BlockSpec((128, 128), lambda i: (0, 0)),
            pl.BlockSpec((1, 128), lambda i: (0, 0)),
        ],
        out_specs=pl.BlockSpec((bn, 128), lambda i: (i, 0)),
        out_shape=jax.ShapeDtypeStruct((n_pad, 128), jnp.float32),
    )(h_pad, a0, a1, wr, wn, b_row)


def _dense_pool_tc(h_pad, a0, a1, wr, wn, b_row, batch2d, n_pad, bn):
    grid = n_pad // bn

    def body(h_ref, a0_ref, a1_ref, wr_ref, wn_ref, b_ref, bt_ref,
             o_ref, ge_ref, sums, counts):
        i = pl.program_id(0)
        z = jnp.dot(h_ref[...], wr_ref[...],
                    preferred_element_type=jnp.float32,
                    precision=lax.Precision.HIGHEST)
        z += jnp.dot(a0_ref[...] + a1_ref[...], wn_ref[...],
                     preferred_element_type=jnp.float32,
                     precision=lax.Precision.HIGHEST)
        z = jnp.maximum(z + b_ref[...], 0.0)
        o_ref[...] = z

        @pl.when(i == 0)
        def _():
            sums[...] = jnp.zeros_like(sums)
            counts[...] = jnp.zeros_like(counts)

        gi = lax.broadcasted_iota(jnp.int32, (G_OUT, bn), 0)
        oh = (bt_ref[...] == gi).astype(jnp.float32)
        sums[...] += jnp.dot(oh, z, preferred_element_type=jnp.float32,
                             precision=lax.Precision.HIGHEST)
        counts[...] += jnp.broadcast_to(
            jnp.sum(oh, axis=1, keepdims=True), (G_OUT, 128))

        @pl.when(i == grid - 1)
        def _():
            ge_ref[...] = sums[...] / jnp.maximum(counts[...], 1.0)

    return pl.pallas_call(
        body,
        grid=(grid,),
        in_specs=[
            pl.BlockSpec((bn, 128), lambda i: (i, 0)),
            pl.BlockSpec((bn, 128), lambda i: (i, 0)),
            pl.BlockSpec((bn, 128), lambda i: (i, 0)),
            pl.BlockSpec((128, 128), lambda i: (0, 0)),
            pl.BlockSpec((128, 128), lambda i: (0, 0)),
            pl.BlockSpec((1, 128), lambda i: (0, 0)),
            pl.BlockSpec((1, bn), lambda i: (0, i)),
        ],
        out_specs=[
            pl.BlockSpec((bn, 128), lambda i: (i, 0)),
            pl.BlockSpec((G_OUT, 128), lambda i: (0, 0)),
        ],
        out_shape=[
            jax.ShapeDtypeStruct((n_pad, 128), jnp.float32),
            jax.ShapeDtypeStruct((G_OUT, 128), jnp.float32),
        ],
        scratch_shapes=[
            pltpu.VMEM((G_OUT, 128), jnp.float32),
            pltpu.VMEM((G_OUT, 128), jnp.float32),
        ],
    )(h_pad, a0, a1, wr, wn, b_row, batch2d)


def kernel(x, edge_index, batch, W_root0, W_nbr0, b0, W_root1, W_nbr1, b1):
    N, D = x.shape
    E = edge_index.shape[1]
    BN = 1024
    n_pad = ((N + 1 + BN - 1) // BN) * BN
    nch = -(-E // (NW * CH))
    e_pad = NW * nch * CH - E

    x_pad = jnp.pad(x, ((0, n_pad - N), (0, 0)))
    src = edge_index[0]
    dst = edge_index[1]
    src_w = jnp.concatenate(
        [src, jnp.zeros((e_pad,), jnp.int32)]).reshape(NW, nch, CH)
    dst_w = jnp.concatenate(
        [dst, jnp.full((e_pad,), N, jnp.int32)]).reshape(NW, nch, CH)
    batch2d = jnp.pad(batch, (0, n_pad - N),
                      constant_values=G_OUT).reshape(1, n_pad)
    zeros_blk = jnp.zeros((n_pad // NS, 128), jnp.float32)
    b0r = b0.reshape(1, D)
    b1r = b1.reshape(1, D)

    a = _scatter_sum_sc(x_pad, src_w, dst_w, zeros_blk, n_pad, nch)
    h1 = _dense_tc(x_pad, a[0], a[1], W_root0, W_nbr0, b0r, n_pad, BN)
    a2 = _scatter_sum_sc(h1, src_w, dst_w, zeros_blk, n_pad, nch)
    h2, ge = _dense_pool_tc(h1, a2[0], a2[1], W_root1, W_nbr1, b1r,
                            batch2d, n_pad, BN)
    return h2[:N], ge
